# Initial kernel scaffold; baseline (speedup 1.0000x reference)
import jax
import jax.numpy as jnp
from jax.experimental import pallas as pl


def kernel(g, h, W1, b1, W2, b2):
    raise NotImplementedError("write your pallas kernel here")


# speedup vs baseline: 15.5742x; 15.5742x over previous
import functools

import jax
import jax.numpy as jnp
from jax import lax
from jax.experimental import pallas as pl
from jax.experimental.pallas import tpu as pltpu
from jax.experimental.pallas import tpu_sc as plsc

N_NODES = 10000
N_EDGES = 320000
DIM = 128

NC, NS = 2, 16
NW = NC * NS
NP = 10240
ROWS_PER_SUB = NP // NS

K = 125
EROWS = N_EDGES // K
AGG_C = EROWS // NW
DEG_C = EROWS // NW
DEG_L = 128
SB = 8
NSB = AGG_C // SB

_mesh = plsc.VectorSubcoreMesh(core_axis_name="c", subcore_axis_name="s")


@functools.partial(
    pl.kernel,
    out_type=jax.ShapeDtypeStruct((NC * NP, DEG_L), jnp.float32),
    mesh=_mesh,
    scratch_types=[
        pltpu.VMEM((SB, K), jnp.int32),
        pltpu.VMEM((K, DEG_L), jnp.float32),
        pltpu.VMEM_SHARED((NP, DEG_L), jnp.float32),
    ],
)
def _sc_deg(dst_hbm, zeros_hbm, ones_hbm, out_hbm, dst_v, ones_v, acc):
    cid = lax.axis_index("c")
    sid = lax.axis_index("s")
    t = cid * NS + sid

    pltpu.sync_copy(zeros_hbm, acc.at[pl.ds(sid * ROWS_PER_SUB, ROWS_PER_SUB)])
    pltpu.sync_copy(ones_hbm, ones_v)
    plsc.subcore_barrier()

    def sb_body(b, carry):
        pltpu.sync_copy(dst_hbm.at[pl.ds(t * DEG_C + b * SB, SB)], dst_v)

        def body(j, c2):
            pltpu.sync_copy(ones_v, acc.at[dst_v.at[j]], add=True)
            return c2

        return lax.fori_loop(0, SB, body, carry)

    lax.fori_loop(0, NSB, sb_body, 0)
    plsc.subcore_barrier()

    pltpu.sync_copy(
        acc.at[pl.ds(sid * ROWS_PER_SUB, ROWS_PER_SUB)],
        out_hbm.at[pl.ds(cid * NP + sid * ROWS_PER_SUB, ROWS_PER_SUB)],
    )


@functools.partial(
    pl.kernel,
    out_type=jax.ShapeDtypeStruct((NC * NP, DIM), jnp.float32),
    mesh=_mesh,
    scratch_types=[
        pltpu.VMEM((SB, K), jnp.int32),
        pltpu.VMEM((SB, K), jnp.int32),
        pltpu.VMEM((K, DIM), jnp.float32),
        pltpu.VMEM((K, DIM), jnp.float32),
        pltpu.VMEM_SHARED((NP, DIM), jnp.float32),
        pltpu.SemaphoreType.DMA,
        pltpu.SemaphoreType.DMA,
    ],
)
def _sc_agg(y_hbm, src_hbm, dst_hbm, zeros_hbm, out_hbm,
            src_v, dst_v, buf_a, buf_b, acc, sem_a, sem_b):
    cid = lax.axis_index("c")
    sid = lax.axis_index("s")
    t = cid * NS + sid

    pltpu.sync_copy(zeros_hbm, acc.at[pl.ds(sid * ROWS_PER_SUB, ROWS_PER_SUB)])
    plsc.subcore_barrier()

    def sb_body(b, carry):
        base = t * AGG_C + b * SB
        pltpu.sync_copy(src_hbm.at[pl.ds(base, SB)], src_v)
        pltpu.sync_copy(dst_hbm.at[pl.ds(base, SB)], dst_v)

        pltpu.async_copy(y_hbm.at[src_v.at[0]], buf_a, sem_a)

        def body(p, c2):
            j = 2 * p
            pltpu.make_async_copy(y_hbm.at[src_v.at[j]], buf_a, sem_a).wait()
            pltpu.async_copy(y_hbm.at[src_v.at[j + 1]], buf_b, sem_b)
            pltpu.sync_copy(buf_a, acc.at[dst_v.at[j]], add=True)
            pltpu.make_async_copy(y_hbm.at[src_v.at[j + 1]], buf_b, sem_b).wait()

            @pl.when(j + 2 < SB)
            def _():
                pltpu.async_copy(y_hbm.at[src_v.at[j + 2]], buf_a, sem_a)

            pltpu.sync_copy(buf_b, acc.at[dst_v.at[j + 1]], add=True)
            return c2

        return lax.fori_loop(0, SB // 2, body, carry)

    lax.fori_loop(0, NSB, sb_body, 0)
    plsc.subcore_barrier()

    pltpu.sync_copy(
        acc.at[pl.ds(sid * ROWS_PER_SUB, ROWS_PER_SUB)],
        out_hbm.at[pl.ds(cid * NP + sid * ROWS_PER_SUB, ROWS_PER_SUB)],
    )


_BR = 1024


def _tc1_body(h_ref, w_ref, b_ref, deg_ref, y_ref, dinv_ref):
    deg = deg_ref[...]
    dsum = deg[:, 0:1] + deg[:, 1:2] + 1.0
    dinv = lax.rsqrt(dsum)
    x = jnp.dot(h_ref[...], w_ref[...], preferred_element_type=jnp.float32)
    y_ref[...] = (x + b_ref[...]) * dinv
    dinv_ref[...] = dinv


def _tc2_body(s_ref, y_ref, dinv_ref, w_ref, b_ref, y2_ref):
    dinv = dinv_ref[...]
    s = s_ref[0] + s_ref[1] + y_ref[...]
    h1 = jnp.maximum(dinv * s, 0.0)
    x2 = jnp.dot(h1, w_ref[...], preferred_element_type=jnp.float32)
    y2_ref[...] = (x2 + b_ref[...]) * dinv


def _tc3_body(s_ref, y_ref, dinv_ref, h0_ref, out_ref):
    s = s_ref[0] + s_ref[1] + y_ref[...]
    h2 = jnp.maximum(dinv_ref[...] * s, 0.0)
    out_ref[...] = jnp.maximum(h2 + h0_ref[...], 0.0)


_row_spec = pl.BlockSpec((_BR, DIM), lambda i: (i, 0))
_w_spec = pl.BlockSpec((DIM, DIM), lambda i: (0, 0))
_b_spec = pl.BlockSpec((1, DIM), lambda i: (0, 0))
_dinv_spec = pl.BlockSpec((_BR, 1), lambda i: (i, 0))
_s_spec = pl.BlockSpec((NC, _BR, DIM), lambda i: (0, i, 0))
_grid = (NP // _BR,)

_tc1 = pl.pallas_call(
    _tc1_body,
    grid=_grid,
    in_specs=[_row_spec, _w_spec, _b_spec, pl.BlockSpec((_BR, 2), lambda i: (i, 0))],
    out_specs=[_row_spec, _dinv_spec],
    out_shape=[
        jax.ShapeDtypeStruct((NP, DIM), jnp.float32),
        jax.ShapeDtypeStruct((NP, 1), jnp.float32),
    ],
)

_tc2 = pl.pallas_call(
    _tc2_body,
    grid=_grid,
    in_specs=[_s_spec, _row_spec, _dinv_spec, _w_spec, _b_spec],
    out_specs=_row_spec,
    out_shape=jax.ShapeDtypeStruct((NP, DIM), jnp.float32),
)

_tc3 = pl.pallas_call(
    _tc3_body,
    grid=_grid,
    in_specs=[_s_spec, _row_spec, _dinv_spec, _row_spec],
    out_specs=_row_spec,
    out_shape=jax.ShapeDtypeStruct((NP, DIM), jnp.float32),
)


def kernel(g, h, W1, b1, W2, b2):
    src2d = g[0].reshape(EROWS, K)
    dst2d = g[1].reshape(EROWS, K)
    h_pad = jnp.pad(h, ((0, NP - N_NODES), (0, 0)))
    zeros_agg = jnp.zeros((ROWS_PER_SUB, DIM), jnp.float32)
    zeros_deg = jnp.zeros((ROWS_PER_SUB, DEG_L), jnp.float32)
    ones_deg = jnp.ones((K, DEG_L), jnp.float32)
    b1r = b1.reshape(1, DIM)
    b2r = b2.reshape(1, DIM)

    deg_raw = _sc_deg(dst2d, zeros_deg, ones_deg)
    deg_pair = deg_raw.reshape(NC, NP, DEG_L)[:, :, 0].transpose(1, 0)

    y1, dinv = _tc1(h_pad, W1, b1r, deg_pair)
    S1 = _sc_agg(y1, src2d, dst2d, zeros_agg).reshape(NC, NP, DIM)
    y2 = _tc2(S1, y1, dinv, W2, b2r)
    S2 = _sc_agg(y2, src2d, dst2d, zeros_agg).reshape(NC, NP, DIM)
    out = _tc3(S2, y2, dinv, h_pad)
    return out[:N_NODES]


# speedup vs baseline: 15.6617x; 1.0056x over previous
import functools

import jax
import jax.numpy as jnp
from jax import lax
from jax.experimental import pallas as pl
from jax.experimental.pallas import tpu as pltpu
from jax.experimental.pallas import tpu_sc as plsc

N_NODES = 10000
N_EDGES = 320000
DIM = 128

NC, NS = 2, 16
NW = NC * NS
NP = 10240
ROWS_PER_SUB = NP // NS

K = 125
EROWS = N_EDGES // K
AGG_C = EROWS // NW
DEG_C = EROWS // NW
DEG_L = 128
SB = 40
NSB = AGG_C // SB

_mesh = plsc.VectorSubcoreMesh(core_axis_name="c", subcore_axis_name="s")


@functools.partial(
    pl.kernel,
    out_type=jax.ShapeDtypeStruct((NC * NP, DEG_L), jnp.float32),
    mesh=_mesh,
    scratch_types=[
        pltpu.VMEM((DEG_C, K), jnp.int32),
        pltpu.VMEM((K, DEG_L), jnp.float32),
        pltpu.VMEM_SHARED((NP, DEG_L), jnp.float32),
        pltpu.SemaphoreType.DMA,
    ],
)
def _sc_deg(dst_hbm, zeros_hbm, ones_hbm, out_hbm, dst_v, ones_v, acc, sem):
    cid = lax.axis_index("c")
    sid = lax.axis_index("s")
    t = cid * NS + sid

    pltpu.sync_copy(zeros_hbm, acc.at[pl.ds(sid * ROWS_PER_SUB, ROWS_PER_SUB)])
    pltpu.sync_copy(ones_hbm, ones_v)
    pltpu.sync_copy(dst_hbm.at[pl.ds(t * DEG_C, DEG_C)], dst_v)
    plsc.subcore_barrier()

    def body(j, carry):
        pltpu.async_copy(ones_v, acc.at[dst_v.at[j]], sem, add=True)

        @pl.when(j >= 4)
        def _():
            pltpu.make_async_copy(ones_v, acc.at[dst_v.at[0]], sem).wait()

        return carry

    lax.fori_loop(0, DEG_C, body, 0)

    def drain(j, carry):
        pltpu.make_async_copy(ones_v, acc.at[dst_v.at[0]], sem).wait()
        return carry

    lax.fori_loop(0, 4, drain, 0)
    plsc.subcore_barrier()

    pltpu.sync_copy(
        acc.at[pl.ds(sid * ROWS_PER_SUB, ROWS_PER_SUB)],
        out_hbm.at[pl.ds(cid * NP + sid * ROWS_PER_SUB, ROWS_PER_SUB)],
    )


@functools.partial(
    pl.kernel,
    out_type=jax.ShapeDtypeStruct((NC * NP, DIM), jnp.float32),
    mesh=_mesh,
    scratch_types=[
        pltpu.VMEM((SB, K), jnp.int32),
        pltpu.VMEM((SB, K), jnp.int32),
        pltpu.VMEM((K, DIM), jnp.float32),
        pltpu.VMEM((K, DIM), jnp.float32),
        pltpu.VMEM_SHARED((NP, DIM), jnp.float32),
        pltpu.SemaphoreType.DMA,
        pltpu.SemaphoreType.DMA,
        pltpu.SemaphoreType.DMA,
        pltpu.SemaphoreType.DMA,
    ],
)
def _sc_agg(y_hbm, src_hbm, dst_hbm, zeros_hbm, out_hbm,
            src_v, dst_v, buf_a, buf_b, acc, sem_ga, sem_gb, sem_sa, sem_sb):
    cid = lax.axis_index("c")
    sid = lax.axis_index("s")
    t = cid * NS + sid

    pltpu.sync_copy(zeros_hbm, acc.at[pl.ds(sid * ROWS_PER_SUB, ROWS_PER_SUB)])
    plsc.subcore_barrier()

    def sb_body(b, carry):
        base = t * AGG_C + b * SB
        pltpu.sync_copy(src_hbm.at[pl.ds(base, SB)], src_v)
        pltpu.sync_copy(dst_hbm.at[pl.ds(base, SB)], dst_v)

        pltpu.async_copy(y_hbm.at[src_v.at[0]], buf_a, sem_ga)
        pltpu.async_copy(y_hbm.at[src_v.at[1]], buf_b, sem_gb)

        def body(p, c2):
            j = 2 * p
            pltpu.make_async_copy(y_hbm.at[src_v.at[j]], buf_a, sem_ga).wait()
            pltpu.async_copy(buf_a, acc.at[dst_v.at[j]], sem_sa, add=True)
            pltpu.make_async_copy(y_hbm.at[src_v.at[j + 1]], buf_b, sem_gb).wait()
            pltpu.async_copy(buf_b, acc.at[dst_v.at[j + 1]], sem_sb, add=True)
            pltpu.make_async_copy(buf_a, acc.at[dst_v.at[j]], sem_sa).wait()

            @pl.when(j + 2 < SB)
            def _():
                pltpu.async_copy(y_hbm.at[src_v.at[j + 2]], buf_a, sem_ga)

            pltpu.make_async_copy(buf_b, acc.at[dst_v.at[j + 1]], sem_sb).wait()

            @pl.when(j + 3 < SB)
            def _():
                pltpu.async_copy(y_hbm.at[src_v.at[j + 3]], buf_b, sem_gb)

            return c2

        return lax.fori_loop(0, SB // 2, body, carry)

    lax.fori_loop(0, NSB, sb_body, 0)
    plsc.subcore_barrier()

    pltpu.sync_copy(
        acc.at[pl.ds(sid * ROWS_PER_SUB, ROWS_PER_SUB)],
        out_hbm.at[pl.ds(cid * NP + sid * ROWS_PER_SUB, ROWS_PER_SUB)],
    )


_BR = 1024


def _tc1_body(h_ref, w_ref, b_ref, deg_ref, y_ref, dinv_ref):
    deg = deg_ref[...]
    dsum = deg[:, 0:1] + deg[:, 1:2] + 1.0
    dinv = lax.rsqrt(dsum)
    x = jnp.dot(h_ref[...], w_ref[...], preferred_element_type=jnp.float32)
    y_ref[...] = (x + b_ref[...]) * dinv
    dinv_ref[...] = dinv


def _tc2_body(s_ref, y_ref, dinv_ref, w_ref, b_ref, y2_ref):
    dinv = dinv_ref[...]
    s = s_ref[0] + s_ref[1] + y_ref[...]
    h1 = jnp.maximum(dinv * s, 0.0)
    x2 = jnp.dot(h1, w_ref[...], preferred_element_type=jnp.float32)
    y2_ref[...] = (x2 + b_ref[...]) * dinv


def _tc3_body(s_ref, y_ref, dinv_ref, h0_ref, out_ref):
    s = s_ref[0] + s_ref[1] + y_ref[...]
    h2 = jnp.maximum(dinv_ref[...] * s, 0.0)
    out_ref[...] = jnp.maximum(h2 + h0_ref[...], 0.0)


_row_spec = pl.BlockSpec((_BR, DIM), lambda i: (i, 0))
_w_spec = pl.BlockSpec((DIM, DIM), lambda i: (0, 0))
_b_spec = pl.BlockSpec((1, DIM), lambda i: (0, 0))
_dinv_spec = pl.BlockSpec((_BR, 1), lambda i: (i, 0))
_s_spec = pl.BlockSpec((NC, _BR, DIM), lambda i: (0, i, 0))
_grid = (NP // _BR,)

_tc1 = pl.pallas_call(
    _tc1_body,
    grid=_grid,
    in_specs=[_row_spec, _w_spec, _b_spec, pl.BlockSpec((_BR, 2), lambda i: (i, 0))],
    out_specs=[_row_spec, _dinv_spec],
    out_shape=[
        jax.ShapeDtypeStruct((NP, DIM), jnp.float32),
        jax.ShapeDtypeStruct((NP, 1), jnp.float32),
    ],
)

_tc2 = pl.pallas_call(
    _tc2_body,
    grid=_grid,
    in_specs=[_s_spec, _row_spec, _dinv_spec, _w_spec, _b_spec],
    out_specs=_row_spec,
    out_shape=jax.ShapeDtypeStruct((NP, DIM), jnp.float32),
)

_tc3 = pl.pallas_call(
    _tc3_body,
    grid=_grid,
    in_specs=[_s_spec, _row_spec, _dinv_spec, _row_spec],
    out_specs=_row_spec,
    out_shape=jax.ShapeDtypeStruct((NP, DIM), jnp.float32),
)


def kernel(g, h, W1, b1, W2, b2):
    src2d = g[0].reshape(EROWS, K)
    dst2d = g[1].reshape(EROWS, K)
    h_pad = jnp.pad(h, ((0, NP - N_NODES), (0, 0)))
    zeros_agg = jnp.zeros((ROWS_PER_SUB, DIM), jnp.float32)
    zeros_deg = jnp.zeros((ROWS_PER_SUB, DEG_L), jnp.float32)
    ones_deg = jnp.ones((K, DEG_L), jnp.float32)
    b1r = b1.reshape(1, DIM)
    b2r = b2.reshape(1, DIM)

    deg_raw = _sc_deg(dst2d, zeros_deg, ones_deg)
    deg_pair = deg_raw.reshape(NC, NP, DEG_L)[:, :, 0].transpose(1, 0)

    y1, dinv = _tc1(h_pad, W1, b1r, deg_pair)
    S1 = _sc_agg(y1, src2d, dst2d, zeros_agg).reshape(NC, NP, DIM)
    y2 = _tc2(S1, y1, dinv, W2, b2r)
    S2 = _sc_agg(y2, src2d, dst2d, zeros_agg).reshape(NC, NP, DIM)
    out = _tc3(S2, y2, dinv, h_pad)
    return out[:N_NODES]
